# baseline (speedup 1.0000x reference)
import dataclasses
import functools

import jax
import jax.numpy as jnp
from jax import lax
from jax.experimental import pallas as pl
from jax.experimental.pallas import tpu as pltpu
from jax.experimental.pallas import tpu_sc as plsc

NC = 2
NS = 16
L = 16
NW = NC * NS
K = 128


def _tc_head_body(x_ref, w_ref, as_ref, ad_ref, h_ref, aso_ref, ado_ref, m_ref):
    x = x_ref[...]
    h = jnp.dot(x, w_ref[...], preferred_element_type=jnp.float32)
    npad = x.shape[0]
    h_ref[...] = h
    h3 = h.reshape(npad // 128, 128, 128)
    asv = jnp.sum(h3 * as_ref[...].reshape(1, 1, 128), axis=-1)
    adv = jnp.sum(h3 * ad_ref[...].reshape(1, 1, 128), axis=-1)
    aso_ref[...] = asv
    ado_ref[...] = adv
    m = jnp.max(asv) + jnp.max(adv)
    m_ref[...] = jnp.full((8, 128), m, jnp.float32)


def _tc_head(x_pad, W, a_s, a_d):
    npad, d = x_pad.shape
    outs = pl.pallas_call(
        _tc_head_body,
        out_shape=[
            jax.ShapeDtypeStruct((npad, d), jnp.float32),
            jax.ShapeDtypeStruct((npad // 128, 128), jnp.float32),
            jax.ShapeDtypeStruct((npad // 128, 128), jnp.float32),
            jax.ShapeDtypeStruct((8, 128), jnp.float32),
        ],
    )(x_pad, W, a_s.reshape(1, d), a_d.reshape(1, d))
    h, asv, adv, m = outs
    return h, asv.reshape(-1), adv.reshape(-1), m.reshape(-1)[:L]


def _tc_norm_body(n_real, num_ref, den_ref, b_ref, o_ref):
    npad = num_ref.shape[1]
    nr = npad // 128
    p3 = (num_ref[0] + num_ref[1]).reshape(nr, 128, 128)
    den = jnp.sum(den_ref[...], axis=0)
    y = p3 / (den[:, :, None] + 1e-16) + b_ref[...].reshape(1, 1, 128)
    y = jnp.maximum(y, 0.0)
    r0 = lax.broadcasted_iota(jnp.int32, (nr, 128, 1), 0)
    r1 = lax.broadcasted_iota(jnp.int32, (nr, 128, 1), 1)
    y = jnp.where(r0 * 128 + r1 < n_real, y, 0.0)
    o_ref[...] = y.reshape(npad, 128)


def _tc_norm_head_body(n_real, num_ref, den_ref, b_ref, w_ref, as_ref,
                       ad_ref, h_ref, aso_ref, ado_ref, m_ref):
    npad = num_ref.shape[1]
    nr = npad // 128
    p3 = (num_ref[0] + num_ref[1]).reshape(nr, 128, 128)
    den = jnp.sum(den_ref[...], axis=0)
    y = p3 / (den[:, :, None] + 1e-16) + b_ref[...].reshape(1, 1, 128)
    y = jnp.maximum(y, 0.0)
    r0 = lax.broadcasted_iota(jnp.int32, (nr, 128, 1), 0)
    r1 = lax.broadcasted_iota(jnp.int32, (nr, 128, 1), 1)
    y = jnp.where(r0 * 128 + r1 < n_real, y, 0.0)
    x2 = y.reshape(npad, 128)
    h = jnp.dot(x2, w_ref[...], preferred_element_type=jnp.float32)
    h_ref[...] = h
    h3 = h.reshape(nr, 128, 128)
    asv = jnp.sum(h3 * as_ref[...].reshape(1, 1, 128), axis=-1)
    adv = jnp.sum(h3 * ad_ref[...].reshape(1, 1, 128), axis=-1)
    aso_ref[...] = asv
    ado_ref[...] = adv
    m = jnp.max(asv) + jnp.max(adv)
    m_ref[...] = jnp.full((8, 128), m, jnp.float32)


def _tc_norm_head(part, den, bias, W, a_s, a_d, n_real):
    npad = part.shape[1]
    d = W.shape[0]
    outs = pl.pallas_call(
        functools.partial(_tc_norm_head_body, n_real),
        out_shape=[
            jax.ShapeDtypeStruct((npad, d), jnp.float32),
            jax.ShapeDtypeStruct((npad // 128, 128), jnp.float32),
            jax.ShapeDtypeStruct((npad // 128, 128), jnp.float32),
            jax.ShapeDtypeStruct((8, 128), jnp.float32),
        ],
    )(part, den.reshape(NC * NS, npad // 128, 128), bias.reshape(1, 128),
      W, a_s.reshape(1, d), a_d.reshape(1, d))
    h, asv, adv, m = outs
    return h, asv.reshape(-1), adv.reshape(-1), m.reshape(-1)[:L]


def _tc_norm(part, den, bias, n_real):
    npad = part.shape[1]
    return pl.pallas_call(
        functools.partial(_tc_norm_body, n_real),
        out_shape=jax.ShapeDtypeStruct((npad, 128), jnp.float32),
    )(part, den.reshape(NC * NS, npad // 128, 128), bias.reshape(1, 128))


def _sc_w_body(nb, as_hbm, ad_hbm, m_hbm, src_hbm, dst_hbm, w_hbm, den_hbm,
               as_v, ad_v, m_v, src_v, dst_v, w_buf, den_v):
    cid = lax.axis_index("c")
    sid = lax.axis_index("s")
    wid = sid * NC + cid

    pltpu.sync_copy(as_hbm, as_v)
    pltpu.sync_copy(ad_hbm, ad_v)
    pltpu.sync_copy(m_hbm, m_v)
    pltpu.sync_copy(src_hbm.at[wid], src_v)
    pltpu.sync_copy(dst_hbm.at[wid], dst_v)

    zero16 = jnp.zeros((L,), jnp.float32)

    @pl.loop(0, den_v.shape[0])
    def _z(i):
        for c in range(den_v.shape[1] // L):
            den_v[i, pl.ds(c * L, L)] = zero16

    mv = m_v[...]
    lane_iota = lax.broadcasted_iota(jnp.int32, (L,), 0)

    @pl.loop(0, nb)
    def _blk(b):
        @pl.loop(0, K // L)
        def _grp(g):
            sidx = src_v[b, pl.ds(g * L, L)]
            didx = dst_v[b, pl.ds(g * L, L)]
            s = plsc.load_gather(as_v, [sidx])
            t = plsc.load_gather(ad_v, [didx])
            e = s + t
            e = jnp.where(e > 0, e, e * jnp.float32(0.2))
            w = jnp.exp(e - mv)
            w_buf[b, pl.ds(g * L, L)] = w
            drow = jax.lax.shift_right_logical(didx, 7)
            dcol = jnp.bitwise_and(didx, 127)
            for j in range(L):
                plsc.addupdate_scatter(den_v, [drow, dcol], w,
                                       mask=lane_iota == j)

    pltpu.sync_copy(w_buf, w_hbm.at[wid])
    pltpu.sync_copy(den_v, den_hbm.at[cid, sid])


def _sc_a_body(nb, nbc, h_hbm, src_hbm, dst_hbm, w_hbm, zeros_hbm, out_hbm,
               src_c, dst_c, w_c, r0, r1, out_sp, sem0, sem1):
    cid = lax.axis_index("c")
    sid = lax.axis_index("s")
    wid = sid * NC + cid
    nacc = out_sp.shape[0]
    zps = nacc // NS
    d = r0.shape[1]

    pltpu.sync_copy(zeros_hbm.at[cid, pl.ds(sid * zps, zps)],
                    out_sp.at[pl.ds(sid * zps, zps)])
    plsc.subcore_barrier()

    def scale_and_scatter(buf, bi):
        bidx = lax.broadcast(bi, (L,))

        @plsc.parallel_loop(0, K, unroll=8)
        def _row(r):
            wv = plsc.load_gather(w_c, [bidx, lax.broadcast(r, (L,))])
            for c in range(d // L):
                buf[r, pl.ds(c * L, L)] = buf[r, pl.ds(c * L, L)] * wv

        pltpu.sync_copy(buf, out_sp.at[dst_c.at[bi]], add=True)

    @pl.loop(0, nb // nbc)
    def _chunk(ch):
        pltpu.sync_copy(src_hbm.at[wid, ch], src_c)
        pltpu.sync_copy(dst_hbm.at[wid, ch], dst_c)
        pltpu.sync_copy(w_hbm.at[wid, ch], w_c)
        pltpu.async_copy(h_hbm.at[src_c.at[0]], r0, sem0)

        @pl.loop(0, nbc // 2)
        def _pair(p):
            b0 = 2 * p
            pltpu.async_copy(h_hbm.at[src_c.at[b0 + 1]], r1, sem1)
            pltpu.make_async_copy(h_hbm.at[src_c.at[b0]], r0, sem0).wait()
            scale_and_scatter(r0, b0)
            pltpu.async_copy(h_hbm.at[src_c.at[b0 + 2]], r0, sem0)
            pltpu.make_async_copy(h_hbm.at[src_c.at[b0 + 1]], r1, sem1).wait()
            scale_and_scatter(r1, b0 + 1)

        pltpu.make_async_copy(h_hbm.at[src_c.at[nbc - 1]], r0, sem0).wait()
        scale_and_scatter(r0, nbc - 1)

    plsc.subcore_barrier()
    pltpu.sync_copy(out_sp.at[pl.ds(sid * zps, zps)],
                    out_hbm.at[cid, pl.ds(sid * zps, zps)])


def _sc_compiler_params():
    cp = pltpu.CompilerParams()
    if "needs_layout_passes" in pltpu.CompilerParams.__dataclass_fields__:
        cp = dataclasses.replace(cp, needs_layout_passes=False)
    return cp


@functools.lru_cache(maxsize=None)
def _sc_w_kernel(npad, nb):
    mesh = plsc.VectorSubcoreMesh(core_axis_name="c", subcore_axis_name="s")
    return pl.kernel(
        functools.partial(_sc_w_body, nb),
        out_type=[
            jax.ShapeDtypeStruct((NW, nb, K), jnp.float32),
            jax.ShapeDtypeStruct((NC, NS, npad // 128, 128), jnp.float32),
        ],
        mesh=mesh,
        scratch_types=[
            pltpu.VMEM((npad,), jnp.float32),
            pltpu.VMEM((npad,), jnp.float32),
            pltpu.VMEM((L,), jnp.float32),
            pltpu.VMEM((nb, K), jnp.int32),
            pltpu.VMEM((nb, K), jnp.int32),
            pltpu.VMEM((nb, K), jnp.float32),
            pltpu.VMEM((npad // 128, 128), jnp.float32),
        ],
        compiler_params=_sc_compiler_params(),
    )


@functools.lru_cache(maxsize=None)
def _sc_a_kernel(npad, d, nb, nbc):
    mesh = plsc.VectorSubcoreMesh(core_axis_name="c", subcore_axis_name="s")
    return pl.kernel(
        functools.partial(_sc_a_body, nb, nbc),
        out_type=jax.ShapeDtypeStruct((NC, npad, d), jnp.float32),
        mesh=mesh,
        scratch_types=[
            pltpu.VMEM((nbc, K), jnp.int32),
            pltpu.VMEM((nbc, K), jnp.int32),
            pltpu.VMEM((nbc, K), jnp.float32),
            pltpu.VMEM((K, d), jnp.float32),
            pltpu.VMEM((K, d), jnp.float32),
            pltpu.VMEM_SHARED((npad, d), jnp.float32),
            pltpu.SemaphoreType.DMA,
            pltpu.SemaphoreType.DMA,
        ],
        compiler_params=_sc_compiler_params(),
    )


def _sc_gat_edges(h, as_flat, ad_flat, mv, src3, dst3, zeros_out):
    npad, d = h.shape
    nb = src3.shape[1]
    nbc = nb // 3
    w3, den = _sc_w_kernel(npad, nb)(as_flat, ad_flat, mv, src3, dst3)
    src4 = src3.reshape(NW, nb // nbc, nbc, K)
    dst4 = dst3.reshape(NW, nb // nbc, nbc, K)
    w4 = w3.reshape(NW, nb // nbc, nbc, K)
    part = _sc_a_kernel(npad, d, nb, nbc)(h, src4, dst4, w4, zeros_out)
    return part, den.reshape(NC, NS, npad)


def kernel(x, edge_index, W1, a1_src, a1_dst, b1, W2, a2_src, a2_dst, b2):
    n, d = x.shape
    e = edge_index.shape[1]
    npad = ((n + 255) // 256) * 256
    etot = e + n
    nb = -(-etot // (NW * K))
    epad = NW * K * nb

    loop = jnp.arange(n, dtype=edge_index.dtype)
    src = jnp.concatenate([edge_index[0], loop,
                           jnp.zeros((epad - etot,), edge_index.dtype)])
    dst = jnp.concatenate([edge_index[1], loop,
                           jnp.full((epad - etot,), n, edge_index.dtype)])
    src3 = src.reshape(NW, nb, K)
    dst3 = dst.reshape(NW, nb, K)
    zeros_out = jnp.zeros((NC, npad, d), jnp.float32)
    x_pad = jnp.zeros((npad, d), jnp.float32).at[:n].set(x)

    h1, as1, ad1, m1 = _tc_head(x_pad, W1, a1_src, a1_dst)
    part1, den1 = _sc_gat_edges(h1, as1, ad1, m1, src3, dst3, zeros_out)
    h2, as2, ad2, m2 = _tc_norm_head(part1, den1, b1, W2, a2_src, a2_dst, n)
    part2, den2 = _sc_gat_edges(h2, as2, ad2, m2, src3, dst3, zeros_out)
    y = _tc_norm(part2, den2, b2, n)

    out = jnp.concatenate([jnp.zeros((1, d), jnp.float32), y[:n]], axis=0)
    return out.reshape(1, n + 1, d)


# speedup vs baseline: 1.0112x; 1.0112x over previous
import dataclasses
import functools

import jax
import jax.numpy as jnp
from jax import lax
from jax.experimental import pallas as pl
from jax.experimental.pallas import tpu as pltpu
from jax.experimental.pallas import tpu_sc as plsc

NC = 2
NS = 16
L = 16
NW = NC * NS
K = 128


def _tc_head_body(x_ref, w_ref, as_ref, ad_ref, h_ref, aso_ref, ado_ref, m_ref):
    x = x_ref[...]
    h = jnp.dot(x, w_ref[...], preferred_element_type=jnp.float32)
    npad = x.shape[0]
    h_ref[...] = h
    h3 = h.reshape(npad // 128, 128, 128)
    asv = jnp.sum(h3 * as_ref[...].reshape(1, 1, 128), axis=-1)
    adv = jnp.sum(h3 * ad_ref[...].reshape(1, 1, 128), axis=-1)
    aso_ref[...] = asv
    ado_ref[...] = adv
    m = jnp.max(asv) + jnp.max(adv)
    m_ref[...] = jnp.full((8, 128), m, jnp.float32)


def _tc_head(x_pad, W, a_s, a_d):
    npad, d = x_pad.shape
    outs = pl.pallas_call(
        _tc_head_body,
        out_shape=[
            jax.ShapeDtypeStruct((npad, d), jnp.float32),
            jax.ShapeDtypeStruct((npad // 128, 128), jnp.float32),
            jax.ShapeDtypeStruct((npad // 128, 128), jnp.float32),
            jax.ShapeDtypeStruct((8, 128), jnp.float32),
        ],
    )(x_pad, W, a_s.reshape(1, d), a_d.reshape(1, d))
    h, asv, adv, m = outs
    return h, asv.reshape(-1), adv.reshape(-1), m.reshape(-1)[:L]


def _tc_norm_body(n_real, num_ref, den_ref, b_ref, o_ref):
    npad = num_ref.shape[1]
    nr = npad // 128
    p3 = (num_ref[0] + num_ref[1]).reshape(nr, 128, 128)
    den = jnp.sum(den_ref[...], axis=0)
    y = p3 / (den[:, :, None] + 1e-16) + b_ref[...].reshape(1, 1, 128)
    y = jnp.maximum(y, 0.0)
    r0 = lax.broadcasted_iota(jnp.int32, (nr, 128, 1), 0)
    r1 = lax.broadcasted_iota(jnp.int32, (nr, 128, 1), 1)
    y = jnp.where(r0 * 128 + r1 < n_real, y, 0.0)
    o_ref[...] = y.reshape(npad, 128)


def _tc_norm_head_body(n_real, num_ref, den_ref, b_ref, w_ref, as_ref,
                       ad_ref, h_ref, aso_ref, ado_ref, m_ref):
    npad = num_ref.shape[1]
    nr = npad // 128
    p3 = (num_ref[0] + num_ref[1]).reshape(nr, 128, 128)
    den = jnp.sum(den_ref[...], axis=0)
    y = p3 / (den[:, :, None] + 1e-16) + b_ref[...].reshape(1, 1, 128)
    y = jnp.maximum(y, 0.0)
    r0 = lax.broadcasted_iota(jnp.int32, (nr, 128, 1), 0)
    r1 = lax.broadcasted_iota(jnp.int32, (nr, 128, 1), 1)
    y = jnp.where(r0 * 128 + r1 < n_real, y, 0.0)
    x2 = y.reshape(npad, 128)
    h = jnp.dot(x2, w_ref[...], preferred_element_type=jnp.float32)
    h_ref[...] = h
    h3 = h.reshape(nr, 128, 128)
    asv = jnp.sum(h3 * as_ref[...].reshape(1, 1, 128), axis=-1)
    adv = jnp.sum(h3 * ad_ref[...].reshape(1, 1, 128), axis=-1)
    aso_ref[...] = asv
    ado_ref[...] = adv
    m = jnp.max(asv) + jnp.max(adv)
    m_ref[...] = jnp.full((8, 128), m, jnp.float32)


def _tc_norm_head(part, den, bias, W, a_s, a_d, n_real):
    npad = part.shape[1]
    d = W.shape[0]
    outs = pl.pallas_call(
        functools.partial(_tc_norm_head_body, n_real),
        out_shape=[
            jax.ShapeDtypeStruct((npad, d), jnp.float32),
            jax.ShapeDtypeStruct((npad // 128, 128), jnp.float32),
            jax.ShapeDtypeStruct((npad // 128, 128), jnp.float32),
            jax.ShapeDtypeStruct((8, 128), jnp.float32),
        ],
    )(part, den.reshape(NC * NS, npad // 128, 128), bias.reshape(1, 128),
      W, a_s.reshape(1, d), a_d.reshape(1, d))
    h, asv, adv, m = outs
    return h, asv.reshape(-1), adv.reshape(-1), m.reshape(-1)[:L]


def _tc_norm(part, den, bias, n_real):
    npad = part.shape[1]
    return pl.pallas_call(
        functools.partial(_tc_norm_body, n_real),
        out_shape=jax.ShapeDtypeStruct((npad, 128), jnp.float32),
    )(part, den.reshape(NC * NS, npad // 128, 128), bias.reshape(1, 128))


def _sc_w_body(nb, as_hbm, ad_hbm, m_hbm, src_hbm, dst_hbm, w_hbm, den_hbm,
               as_v, ad_v, m_v, src_v, dst_v, w_buf, den_v):
    cid = lax.axis_index("c")
    sid = lax.axis_index("s")
    wid = sid * NC + cid

    pltpu.sync_copy(as_hbm, as_v)
    pltpu.sync_copy(ad_hbm, ad_v)
    pltpu.sync_copy(m_hbm, m_v)
    pltpu.sync_copy(src_hbm.at[wid], src_v)
    pltpu.sync_copy(dst_hbm.at[wid], dst_v)

    zero16 = jnp.zeros((L,), jnp.float32)

    @pl.loop(0, den_v.shape[0])
    def _z(i):
        for c in range(den_v.shape[1] // L):
            den_v[i, pl.ds(c * L, L)] = zero16

    mv = m_v[...]
    lane_iota = lax.broadcasted_iota(jnp.int32, (L,), 0)

    @plsc.parallel_loop(0, nb, unroll=2)
    def _blk(b):
        @pl.loop(0, K // L)
        def _grp(g):
            sidx = src_v[b, pl.ds(g * L, L)]
            didx = dst_v[b, pl.ds(g * L, L)]
            s = plsc.load_gather(as_v, [sidx])
            t = plsc.load_gather(ad_v, [didx])
            e = s + t
            e = jnp.where(e > 0, e, e * jnp.float32(0.2))
            w_buf[b, pl.ds(g * L, L)] = jnp.exp(e - mv)

    @pl.loop(0, nb)
    def _dblk(b):
        @pl.loop(0, K // L)
        def _dgrp(g):
            didx = dst_v[b, pl.ds(g * L, L)]
            w = w_buf[b, pl.ds(g * L, L)]
            drow = jax.lax.shift_right_logical(didx, 7)
            dcol = jnp.bitwise_and(didx, 127)
            for j in range(L):
                plsc.addupdate_scatter(den_v, [drow, dcol], w,
                                       mask=lane_iota == j)

    pltpu.sync_copy(w_buf, w_hbm.at[wid])
    pltpu.sync_copy(den_v, den_hbm.at[cid, sid])


def _sc_a_body(nb, nbc, h_hbm, src_hbm, dst_hbm, w_hbm, zeros_hbm, out_hbm,
               src_c, dst_c, w_c, r0, r1, out_sp, sem0, sem1):
    cid = lax.axis_index("c")
    sid = lax.axis_index("s")
    wid = sid * NC + cid
    nacc = out_sp.shape[0]
    zps = nacc // NS
    d = r0.shape[1]

    pltpu.sync_copy(zeros_hbm.at[pl.ds(sid * zps, zps)],
                    out_sp.at[pl.ds(sid * zps, zps)])
    plsc.subcore_barrier()

    def scale_and_scatter(buf, bi):
        bidx = lax.broadcast(bi, (L,))

        @plsc.parallel_loop(0, K, unroll=8)
        def _row(r):
            wv = plsc.load_gather(w_c, [bidx, lax.broadcast(r, (L,))])
            for c in range(d // L):
                buf[r, pl.ds(c * L, L)] = buf[r, pl.ds(c * L, L)] * wv

        pltpu.sync_copy(buf, out_sp.at[dst_c.at[bi]], add=True)

    @pl.loop(0, nb // nbc)
    def _chunk(ch):
        pltpu.sync_copy(src_hbm.at[wid, ch], src_c)
        pltpu.sync_copy(dst_hbm.at[wid, ch], dst_c)
        pltpu.sync_copy(w_hbm.at[wid, ch], w_c)
        pltpu.async_copy(h_hbm.at[src_c.at[0]], r0, sem0)

        @pl.loop(0, nbc // 2)
        def _pair(p):
            b0 = 2 * p
            pltpu.async_copy(h_hbm.at[src_c.at[b0 + 1]], r1, sem1)
            pltpu.make_async_copy(h_hbm.at[src_c.at[b0]], r0, sem0).wait()
            scale_and_scatter(r0, b0)
            pltpu.async_copy(h_hbm.at[src_c.at[b0 + 2]], r0, sem0)
            pltpu.make_async_copy(h_hbm.at[src_c.at[b0 + 1]], r1, sem1).wait()
            scale_and_scatter(r1, b0 + 1)

        pltpu.make_async_copy(h_hbm.at[src_c.at[nbc - 1]], r0, sem0).wait()
        scale_and_scatter(r0, nbc - 1)

    plsc.subcore_barrier()
    pltpu.sync_copy(out_sp.at[pl.ds(sid * zps, zps)],
                    out_hbm.at[cid, pl.ds(sid * zps, zps)])


def _sc_compiler_params():
    cp = pltpu.CompilerParams()
    if "needs_layout_passes" in pltpu.CompilerParams.__dataclass_fields__:
        cp = dataclasses.replace(cp, needs_layout_passes=False)
    return cp


@functools.lru_cache(maxsize=None)
def _sc_w_kernel(npad, nb):
    mesh = plsc.VectorSubcoreMesh(core_axis_name="c", subcore_axis_name="s")
    return pl.kernel(
        functools.partial(_sc_w_body, nb),
        out_type=[
            jax.ShapeDtypeStruct((NW, nb, K), jnp.float32),
            jax.ShapeDtypeStruct((NC, NS, npad // 128, 128), jnp.float32),
        ],
        mesh=mesh,
        scratch_types=[
            pltpu.VMEM((npad,), jnp.float32),
            pltpu.VMEM((npad,), jnp.float32),
            pltpu.VMEM((L,), jnp.float32),
            pltpu.VMEM((nb, K), jnp.int32),
            pltpu.VMEM((nb, K), jnp.int32),
            pltpu.VMEM((nb, K), jnp.float32),
            pltpu.VMEM((npad // 128, 128), jnp.float32),
        ],
        compiler_params=_sc_compiler_params(),
    )


@functools.lru_cache(maxsize=None)
def _sc_a_kernel(npad, d, nb, nbc):
    mesh = plsc.VectorSubcoreMesh(core_axis_name="c", subcore_axis_name="s")
    return pl.kernel(
        functools.partial(_sc_a_body, nb, nbc),
        out_type=jax.ShapeDtypeStruct((NC, npad, d), jnp.float32),
        mesh=mesh,
        scratch_types=[
            pltpu.VMEM((nbc, K), jnp.int32),
            pltpu.VMEM((nbc, K), jnp.int32),
            pltpu.VMEM((nbc, K), jnp.float32),
            pltpu.VMEM((K, d), jnp.float32),
            pltpu.VMEM((K, d), jnp.float32),
            pltpu.VMEM_SHARED((npad, d), jnp.float32),
            pltpu.SemaphoreType.DMA,
            pltpu.SemaphoreType.DMA,
        ],
        compiler_params=_sc_compiler_params(),
    )


def _sc_gat_edges(h, as_flat, ad_flat, mv, src3, dst3, zeros_out):
    npad, d = h.shape
    nb = src3.shape[1]
    nbc = nb // 3
    w3, den = _sc_w_kernel(npad, nb)(as_flat, ad_flat, mv, src3, dst3)
    src4 = src3.reshape(NW, nb // nbc, nbc, K)
    dst4 = dst3.reshape(NW, nb // nbc, nbc, K)
    w4 = w3.reshape(NW, nb // nbc, nbc, K)
    part = _sc_a_kernel(npad, d, nb, nbc)(h, src4, dst4, w4, zeros_out)
    return part, den.reshape(NC, NS, npad)


def kernel(x, edge_index, W1, a1_src, a1_dst, b1, W2, a2_src, a2_dst, b2):
    n, d = x.shape
    e = edge_index.shape[1]
    npad = ((n + 255) // 256) * 256
    etot = e + n
    nb = -(-etot // (NW * K))
    epad = NW * K * nb

    loop = jnp.arange(n, dtype=edge_index.dtype)
    src = jnp.concatenate([edge_index[0], loop,
                           jnp.zeros((epad - etot,), edge_index.dtype)])
    dst = jnp.concatenate([edge_index[1], loop,
                           jnp.full((epad - etot,), n, edge_index.dtype)])
    src3 = src.reshape(NW, nb, K)
    dst3 = dst.reshape(NW, nb, K)
    zeros_out = jnp.zeros((npad, d), jnp.float32)
    x_pad = jnp.zeros((npad, d), jnp.float32).at[:n].set(x)

    h1, as1, ad1, m1 = _tc_head(x_pad, W1, a1_src, a1_dst)
    part1, den1 = _sc_gat_edges(h1, as1, ad1, m1, src3, dst3, zeros_out)
    h2, as2, ad2, m2 = _tc_norm_head(part1, den1, b1, W2, a2_src, a2_dst, n)
    part2, den2 = _sc_gat_edges(h2, as2, ad2, m2, src3, dst3, zeros_out)
    y = _tc_norm(part2, den2, b2, n)

    out = jnp.concatenate([jnp.zeros((1, d), jnp.float32), y[:n]], axis=0)
    return out.reshape(1, n + 1, d)
